# baseline (speedup 1.0000x reference)
import functools

import jax
import jax.numpy as jnp
from jax import lax
from jax.experimental import pallas as pl
from jax.experimental.pallas import tpu as pltpu
from jax.experimental.pallas import tpu_sc as plsc

D = 128
N_ROWS = 100000
NC, NS = 2, 16
NW = NC * NS
ROWS_PER_TILE = N_ROWS // NW
CHUNK_ROWS = 125
NCHUNKS = ROWS_PER_TILE // CHUNK_ROWS
CHUNK_F = CHUNK_ROWS * D
NLANE = 8


def _sc_partial_sums(nbr_flat):
    mesh = plsc.VectorSubcoreMesh(
        core_axis_name="c", subcore_axis_name="s", num_cores=NC, num_subcores=NS
    )

    @functools.partial(
        pl.kernel,
        out_type=jax.ShapeDtypeStruct((NW, D), jnp.float32),
        mesh=mesh,
        scratch_types=[
            pltpu.VMEM((CHUNK_F,), jnp.float32),
            pltpu.VMEM((D,), jnp.float32),
        ],
    )
    def k(nbr_hbm, part_hbm, buf, accv):
        cid = lax.axis_index("c")
        sid = lax.axis_index("s")
        wid = sid * NC + cid
        base = wid * (ROWS_PER_TILE * D)

        def chunk_body(c, accs):
            pltpu.sync_copy(nbr_hbm.at[pl.ds(base + c * CHUNK_F, CHUNK_F)], buf)

            def row_body(r, accs):
                off = r * D
                return tuple(
                    a + buf[pl.ds(off + k * 16, 16)] for k, a in enumerate(accs)
                )

            return lax.fori_loop(0, CHUNK_ROWS, row_body, accs)

        accs = tuple(jnp.zeros((16,), jnp.float32) for _ in range(NLANE))
        accs = lax.fori_loop(0, NCHUNKS, chunk_body, accs)
        for k_i in range(NLANE):
            accv[pl.ds(k_i * 16, 16)] = accs[k_i]
        pltpu.sync_copy(accv, part_hbm.at[wid])

    return k(nbr_flat)


def _tc_finish_body(part_ref, node_ref, w0t_ref, b0_ref, w1t_ref, b1_ref, out_ref):
    mean = jnp.sum(part_ref[...], axis=0, keepdims=True) * (1.0 / N_ROWS)
    node = node_ref[...]
    h = (
        jnp.dot(node, w0t_ref[:D, :], preferred_element_type=jnp.float32)
        + jnp.dot(mean, w0t_ref[D:, :], preferred_element_type=jnp.float32)
        + b0_ref[...]
    )
    h = jnp.maximum(h, 0.0)
    out = jnp.dot(h, w1t_ref[...], preferred_element_type=jnp.float32) + b1_ref[...]
    out_ref[...] = jnp.maximum(out, 0.0)


def _tc_finish(part, node2, w0t, b02, w1t, b12):
    return pl.pallas_call(
        _tc_finish_body,
        out_shape=jax.ShapeDtypeStruct((1, D), jnp.float32),
    )(part, node2, w0t, b02, w1t, b12)


def kernel(node_features, neighbor_features_list, W0, b0, W1, b1):
    nbr_flat = neighbor_features_list.reshape(-1)
    part = _sc_partial_sums(nbr_flat)
    out = _tc_finish(
        part,
        node_features.reshape(1, D),
        W0.T,
        b0.reshape(1, -1),
        W1.T,
        b1.reshape(1, -1),
    )
    return out.reshape(D)


# speedup vs baseline: 1.6603x; 1.6603x over previous
import functools

import jax
import jax.numpy as jnp
from jax import lax
from jax.experimental import pallas as pl
from jax.experimental.pallas import tpu as pltpu
from jax.experimental.pallas import tpu_sc as plsc

D = 128
N_ROWS = 100000
NC, NS = 2, 16
NW = NC * NS
ROWS_PER_TILE = N_ROWS // NW
CHUNK_ROWS = 125
NCHUNKS = ROWS_PER_TILE // CHUNK_ROWS
CHUNK_F = CHUNK_ROWS * D
NLANE = 8
NBUF = 5
NSUPER = NCHUNKS // NBUF


def _sc_partial_sums(nbr_flat):
    mesh = plsc.VectorSubcoreMesh(
        core_axis_name="c", subcore_axis_name="s", num_cores=NC, num_subcores=NS
    )

    @functools.partial(
        pl.kernel,
        out_type=jax.ShapeDtypeStruct((NW, D), jnp.float32),
        mesh=mesh,
        scratch_types=[pltpu.VMEM((CHUNK_F,), jnp.float32)] * NBUF
        + [pltpu.VMEM((D,), jnp.float32)]
        + [pltpu.SemaphoreType.DMA] * NBUF,
    )
    def k(nbr_hbm, part_hbm, *scratch):
        bufs = scratch[:NBUF]
        accv = scratch[NBUF]
        sems = scratch[NBUF + 1 :]
        cid = lax.axis_index("c")
        sid = lax.axis_index("s")
        wid = sid * NC + cid
        base = wid * (ROWS_PER_TILE * D)

        def start(chunk, b):
            pltpu.async_copy(
                nbr_hbm.at[pl.ds(base + chunk * CHUNK_F, CHUNK_F)], bufs[b], sems[b]
            )

        def wait(b):
            pltpu.make_async_copy(
                nbr_hbm.at[pl.ds(base, CHUNK_F)], bufs[b], sems[b]
            ).wait()

        def accumulate(b, accs):
            def row_body(r, accs):
                off = r * D
                return tuple(
                    a + bufs[b][pl.ds(off + k * 16, 16)] for k, a in enumerate(accs)
                )

            return lax.fori_loop(0, CHUNK_ROWS, row_body, accs, unroll=5)

        for b in range(NBUF):
            start(b, b)

        def superchunk(si, accs, fire):
            for b in range(NBUF):
                wait(b)
                accs = accumulate(b, accs)
                if fire:
                    start(si * NBUF + b + NBUF, b)
            return accs

        accs = tuple(jnp.zeros((16,), jnp.float32) for _ in range(NLANE))
        accs = lax.fori_loop(
            0, NSUPER - 1, lambda si, a: superchunk(si, a, True), accs
        )
        accs = superchunk(NSUPER - 1, accs, False)
        for k_i in range(NLANE):
            accv[pl.ds(k_i * 16, 16)] = accs[k_i]
        pltpu.sync_copy(accv, part_hbm.at[wid])

    return k(nbr_flat)


def _tc_finish_body(part_ref, node_ref, w0t_ref, b0_ref, w1t_ref, b1_ref, out_ref):
    mean = jnp.sum(part_ref[...], axis=0, keepdims=True) * (1.0 / N_ROWS)
    node = node_ref[...]
    h = (
        jnp.dot(node, w0t_ref[:D, :], preferred_element_type=jnp.float32)
        + jnp.dot(mean, w0t_ref[D:, :], preferred_element_type=jnp.float32)
        + b0_ref[...]
    )
    h = jnp.maximum(h, 0.0)
    out = jnp.dot(h, w1t_ref[...], preferred_element_type=jnp.float32) + b1_ref[...]
    out_ref[...] = jnp.maximum(out, 0.0)


def _tc_finish(part, node2, w0t, b02, w1t, b12):
    return pl.pallas_call(
        _tc_finish_body,
        out_shape=jax.ShapeDtypeStruct((1, D), jnp.float32),
    )(part, node2, w0t, b02, w1t, b12)


def kernel(node_features, neighbor_features_list, W0, b0, W1, b1):
    nbr_flat = neighbor_features_list.reshape(-1)
    part = _sc_partial_sums(nbr_flat)
    out = _tc_finish(
        part,
        node_features.reshape(1, D),
        W0.T,
        b0.reshape(1, -1),
        W1.T,
        b1.reshape(1, -1),
    )
    return out.reshape(D)
